# baseline (speedup 1.0000x reference)
import jax
import jax.numpy as jnp
import numpy as np
from jax import lax
from jax.experimental import pallas as pl
from jax.experimental.pallas import tpu as pltpu
from jax.experimental.pallas import tpu_sc as plsc

N = 10000
E = 320000
FIN = 128
H = 4
C = 32
HC = H * C
NGRAPH = 64

NCORES = 2
NSUB = 16
EPT = E // NSUB
G = 80
GPT = EPT // G
SCH = 5
NCHUNK = GPT // SCH
NPAD = 10240
HALF = NPAD // NCORES
APC = HALF // NSUB
DPACK = 4
DROWS = HALF // DPACK
DPC = DROWS // NSUB
G2 = G // 2

_f32 = jnp.float32


def _proj_kernel(x_ref, w1_ref, p_ref, xh_ref, asd_ref):
  xh = jnp.dot(x_ref[...].astype(jnp.bfloat16), w1_ref[...].astype(jnp.bfloat16),
               preferred_element_type=_f32)
  xh_ref[...] = xh
  asd_ref[...] = jnp.dot(xh, p_ref[...], preferred_element_type=_f32,
                         precision=lax.Precision.HIGHEST)


def _run_proj(x, W1, P):
  bn = 1000
  return pl.pallas_call(
      _proj_kernel,
      grid=(N // bn,),
      in_specs=[
          pl.BlockSpec((bn, FIN), lambda i: (i, 0)),
          pl.BlockSpec((FIN, HC), lambda i: (0, 0)),
          pl.BlockSpec((HC, 2 * H), lambda i: (0, 0)),
      ],
      out_specs=[
          pl.BlockSpec((bn, HC), lambda i: (i, 0)),
          pl.BlockSpec((bn, 2 * H), lambda i: (i, 0)),
      ],
      out_shape=[
          jax.ShapeDtypeStruct((N, HC), _f32),
          jax.ShapeDtypeStruct((N, 2 * H), _f32),
      ],
  )(x, W1, P)


_LOG2E = 1.4426950408889634
_LN2 = 0.6931471805599453


def _vexp(x):
  xl = x * _LOG2E
  xl = jnp.minimum(jnp.maximum(xl, -125.0), 125.0)
  half = jnp.where(xl >= 0, 0.5, -0.5)
  k = (xl + half).astype(jnp.int32)
  t = (xl - k.astype(_f32)) * _LN2
  p = 1.0 + t * (1.0 + t * (0.5 + t * (
      0.16666666666666666 + t * (0.041666666666666664 + t * (
          0.008333333333333333 + t * 0.001388888888888889)))))
  return p * plsc.bitcast(lax.shift_left(k + 127, 23), _f32)


def _edge_kernel(srcf_hbm, dst4d_hbm, asrc_hbm, adst_hbm, xh_hbm, zeros_hbm,
                 parts_hbm, denp_hbm,
                 asrc_v, adst_v, srcf_v, dst2d_v, ldst_v, didx_v, rows, stag2,
                 agg_sh, den_sh, sem):
  c = lax.axis_index("c")
  s = lax.axis_index("s")
  lo = c * HALF

  pltpu.sync_copy(asrc_hbm, asrc_v)
  pltpu.sync_copy(adst_hbm.at[c], adst_v)
  pltpu.sync_copy(zeros_hbm, agg_sh.at[pl.ds(s * APC, APC)])
  pltpu.sync_copy(zeros_hbm.at[pl.ds(0, DPC)], den_sh.at[pl.ds(s * DPC, DPC)])
  pltpu.sync_copy(zeros_hbm.at[pl.ds(0, 16)], stag2)
  plsc.subcore_barrier()

  iota16 = lax.iota(jnp.int32, 16)

  def group(j, carry):
    jc = j % SCH

    @pl.when(jc == 0)
    def _():
      pltpu.sync_copy(
          srcf_hbm.at[pl.ds(s * EPT + (j // SCH) * (SCH * G), SCH * G)],
          srcf_v)
      pltpu.sync_copy(dst4d_hbm.at[s, j // SCH], dst2d_v)

    gat = pltpu.async_copy(xh_hbm.at[srcf_v.at[pl.ds(jc * G, G)]], rows, sem)

    exs = []
    offs = []
    for t in range(G // 16):
      src16 = srcf_v[pl.ds(jc * G + t * 16, 16)]
      dst16 = dst2d_v[jc, pl.ds(t * 16, 16)]
      inh = (dst16 >= lo) & (dst16 < lo + HALF)
      ldst16 = jnp.where(inh, dst16 - lo, jnp.bitwise_and(dst16, 4095))
      ldst_v[0, pl.ds(t * 16, 16)] = ldst16
      didx_v[t, pl.ds(0, 16)] = lax.shift_right_logical(ldst16, 2)
      offs.append(lax.shift_left(jnp.bitwise_and(ldst16, 3), 2))
      ex_h = []
      for h in range(H):
        av = plsc.load_gather(asrc_v, [src16 * 4 + h])
        bv = plsc.load_gather(adst_v, [ldst16 * 4 + h])
        al = av + bv
        al = jnp.where(al >= 0, al, 0.2 * al)
        ex = _vexp(al)
        ex_h.append(jnp.where(inh, ex, 0.0))
      exs.append(ex_h)

    gat.wait()

    for t in range(G // 16):
      for l in range(16):
        e = t * 16 + l
        wv = [jnp.full((16,), exs[t][h][l]) for h in range(H)]
        off = offs[t][l]
        for k in range(HC // 16):
          rows[e, pl.ds(k * 16, 16)] = \
              rows[e, pl.ds(k * 16, 16)] * wv[k // 2]
        aug = jnp.zeros((16,), _f32)
        for h in range(H):
          aug = jnp.where(iota16 == off + h, wv[h], aug)
        stag2[l, pl.ds(0, 16)] = aug
      pltpu.sync_copy(stag2, den_sh.at[didx_v.at[t]], add=True)
    pltpu.sync_copy(rows, agg_sh.at[ldst_v.at[0]], add=True)
    return carry

  lax.fori_loop(0, GPT, group, 0)
  plsc.subcore_barrier()
  pltpu.sync_copy(agg_sh.at[pl.ds(s * APC, APC)],
                  parts_hbm.at[c, pl.ds(s * APC, APC)])
  pltpu.sync_copy(den_sh.at[pl.ds(s * DPC, DPC)],
                  denp_hbm.at[c, pl.ds(s * DPC, DPC)])


def _run_edges(srcf, dst4d, asrc, adst, xh, zeros):
  mesh = plsc.VectorSubcoreMesh(core_axis_name="c", subcore_axis_name="s")
  fn = pl.kernel(
      _edge_kernel,
      out_type=[
          jax.ShapeDtypeStruct((NCORES, HALF, HC), _f32),
          jax.ShapeDtypeStruct((NCORES, DROWS, HC), _f32),
      ],
      mesh=mesh,
      compiler_params=pltpu.CompilerParams(needs_layout_passes=False),
      scratch_types=[
          pltpu.VMEM((4 * N,), _f32),
          pltpu.VMEM((4 * HALF,), _f32),
          pltpu.VMEM((SCH * G,), jnp.int32),
          pltpu.VMEM((SCH, G), jnp.int32),
          pltpu.VMEM((1, G), jnp.int32),
          pltpu.VMEM((G // 16, 16), jnp.int32),
          pltpu.VMEM((G, HC), _f32),
          pltpu.VMEM((16, HC), _f32),
          pltpu.VMEM_SHARED((HALF, HC), _f32),
          pltpu.VMEM_SHARED((DROWS, HC), _f32),
          pltpu.SemaphoreType.DMA,
      ],
  )
  return fn(srcf, dst4d, asrc, adst, xh, zeros)


def _head_kernel(parts_ref, denp_ref, batch_ref, b1_ref, wd1_ref, bd1_ref,
                 wd2_ref, bd2_ref, out_ref, gmax):
  i = pl.program_id(0)

  @pl.when(i == 0)
  def _():
    gmax[...] = jnp.full((NGRAPH, C), -1e30, _f32)

  agg = parts_ref[0]
  dpk = denp_ref[0]

  per_q = []
  for q in range(DPACK):
    cols = [jnp.broadcast_to(dpk[:, q * H + h:q * H + h + 1], (256, C))
            for h in range(H)]
    per_q.append(jnp.concatenate(cols, axis=1).reshape(256, 1, HC))
  den_rep = jnp.concatenate(per_q, axis=1).reshape(256 * DPACK, HC)

  h1 = agg / (den_rep + 1e-16) + b1_ref[...]
  h1 = jnp.where(h1 >= 0, h1, 0.01 * h1)
  h2 = jnp.dot(h1.astype(jnp.bfloat16), wd1_ref[...].astype(jnp.bfloat16),
               preferred_element_type=_f32) + bd1_ref[...]
  h2 = jnp.where(h2 >= 0, h2, 0.01 * h2)

  b = batch_ref[...]
  for g in range(NGRAPH):
    sel = jnp.where(b == g, h2, -1e30)
    m = jnp.max(sel, axis=0, keepdims=True)
    gmax[g:g + 1, :] = jnp.maximum(gmax[g:g + 1, :], m)

  @pl.when(i == pl.num_programs(0) - 1)
  def _():
    gf = gmax[...]
    gf = jnp.where(gf > -1e29, gf, 0.0)
    out_ref[...] = jnp.dot(gf.astype(jnp.bfloat16),
                           wd2_ref[...].astype(jnp.bfloat16),
                           preferred_element_type=_f32) + bd2_ref[...]


def _run_head(parts, denp, batchp, b1, Wd1, bd1, Wd2, bd2):
  bn = 1024
  nbh = HALF // bn
  return pl.pallas_call(
      _head_kernel,
      grid=(NPAD // bn,),
      in_specs=[
          pl.BlockSpec((1, bn, HC), lambda i: (i // nbh, i % nbh, 0)),
          pl.BlockSpec((1, bn // DPACK, HC), lambda i: (i // nbh, i % nbh, 0)),
          pl.BlockSpec((bn, 1), lambda i: (i, 0)),
          pl.BlockSpec((1, HC), lambda i: (0, 0)),
          pl.BlockSpec((HC, C), lambda i: (0, 0)),
          pl.BlockSpec((1, C), lambda i: (0, 0)),
          pl.BlockSpec((C, 1), lambda i: (0, 0)),
          pl.BlockSpec((1, 1), lambda i: (0, 0)),
      ],
      out_specs=pl.BlockSpec((NGRAPH, 1), lambda i: (0, 0)),
      out_shape=jax.ShapeDtypeStruct((NGRAPH, 1), _f32),
      scratch_shapes=[pltpu.VMEM((NGRAPH, C), _f32)],
  )(parts, denp, batchp, b1, Wd1, bd1, Wd2, bd2)


_RIDX = np.arange(HC)
_HIDX = _RIDX // C


def kernel(x, edge_index, edge_attr, batch, W1, att_src, att_dst, b1, Wd1,
           bd1, Wd2, bd2):
  del edge_attr
  src = edge_index[0]
  dst4d = edge_index[1].reshape(NSUB, NCHUNK, SCH, G)

  P = jnp.zeros((HC, 2 * H), _f32)
  P = P.at[_RIDX, _HIDX].set(att_src.reshape(HC))
  P = P.at[_RIDX, H + _HIDX].set(att_dst.reshape(HC))

  xh, asd = _run_proj(x, W1, P)
  asrc = asd[:, :H].reshape(H * N)
  adst = jnp.concatenate(
      [asd[:, H:], jnp.zeros((NPAD - N, H), _f32)]).reshape(NCORES, H * HALF)
  zeros = jnp.zeros((APC, HC), _f32)
  parts, denp = _run_edges(src, dst4d, asrc, adst, xh, zeros)
  batchp = jnp.concatenate(
      [batch, jnp.full((NPAD - N,), NGRAPH, jnp.int32)]).reshape(NPAD, 1)
  out = _run_head(parts, denp, batchp, b1.reshape(1, HC), Wd1,
                  bd1.reshape(1, C), Wd2, bd2.reshape(1, 1))
  return out


# speedup vs baseline: 1.1777x; 1.1777x over previous
import jax
import jax.numpy as jnp
import numpy as np
from jax import lax
from jax.experimental import pallas as pl
from jax.experimental.pallas import tpu as pltpu
from jax.experimental.pallas import tpu_sc as plsc

N = 10000
E = 320000
FIN = 128
H = 4
C = 32
HC = H * C
NGRAPH = 64

NCORES = 2
NSUB = 16
EPT = E // NSUB
G = 80
GPT = EPT // G
SCH = 5
NCHUNK = GPT // SCH
NPAD = 10240
HALF = NPAD // NCORES
APC = HALF // NSUB
DLEN = HALF * H
DPC = DLEN // NSUB

_f32 = jnp.float32


def _proj_kernel(x_ref, w1_ref, p_ref, xh_ref, asd_ref):
  xh = jnp.dot(x_ref[...].astype(jnp.bfloat16), w1_ref[...].astype(jnp.bfloat16),
               preferred_element_type=_f32)
  xh_ref[...] = xh
  asd_ref[...] = jnp.dot(xh, p_ref[...], preferred_element_type=_f32,
                         precision=lax.Precision.HIGHEST)


def _run_proj(x, W1, P):
  bn = 1000
  return pl.pallas_call(
      _proj_kernel,
      grid=(N // bn,),
      in_specs=[
          pl.BlockSpec((bn, FIN), lambda i: (i, 0)),
          pl.BlockSpec((FIN, HC), lambda i: (0, 0)),
          pl.BlockSpec((HC, 2 * H), lambda i: (0, 0)),
      ],
      out_specs=[
          pl.BlockSpec((bn, HC), lambda i: (i, 0)),
          pl.BlockSpec((bn, 2 * H), lambda i: (i, 0)),
      ],
      out_shape=[
          jax.ShapeDtypeStruct((N, HC), _f32),
          jax.ShapeDtypeStruct((N, 2 * H), _f32),
      ],
  )(x, W1, P)


_LOG2E = 1.4426950408889634
_LN2 = 0.6931471805599453


def _vexp(x):
  xl = x * _LOG2E
  xl = jnp.minimum(jnp.maximum(xl, -125.0), 125.0)
  half = jnp.where(xl >= 0, 0.5, -0.5)
  k = (xl + half).astype(jnp.int32)
  t = (xl - k.astype(_f32)) * _LN2
  p = 1.0 + t * (1.0 + t * (0.5 + t * (
      0.16666666666666666 + t * (0.041666666666666664 + t * (
          0.008333333333333333 + t * 0.001388888888888889)))))
  return p * plsc.bitcast(lax.shift_left(k + 127, 23), _f32)


def _edge_kernel(srcf_hbm, dst4d_hbm, asrc_hbm, adst_hbm, xh_hbm, zeros_hbm,
                 zerosd_hbm, parts_hbm, denp_hbm,
                 asrc_v, adst_v, srcf_v, dst2d_v, ldst_v, den_idx, den_val,
                 rows, agg_sh, den_sh, sem):
  c = lax.axis_index("c")
  s = lax.axis_index("s")
  lo = c * HALF

  pltpu.sync_copy(asrc_hbm, asrc_v)
  pltpu.sync_copy(adst_hbm.at[c], adst_v)
  pltpu.sync_copy(zeros_hbm, agg_sh.at[pl.ds(s * APC, APC)])
  pltpu.sync_copy(zerosd_hbm, den_sh.at[pl.ds(s * DPC, DPC)])
  plsc.subcore_barrier()

  iota16 = lax.iota(jnp.int32, 16)

  def group(j, carry):
    jc = j % SCH

    @pl.when(jc == 0)
    def _():
      pltpu.sync_copy(
          srcf_hbm.at[pl.ds(s * EPT + (j // SCH) * (SCH * G), SCH * G)],
          srcf_v)
      pltpu.sync_copy(dst4d_hbm.at[s, j // SCH], dst2d_v)

    gat = pltpu.async_copy(xh_hbm.at[srcf_v.at[pl.ds(jc * G, G)]], rows, sem)

    exs = []
    for t in range(G // 16):
      src16 = srcf_v[pl.ds(jc * G + t * 16, 16)]
      dst16 = dst2d_v[jc, pl.ds(t * 16, 16)]
      inh = (dst16 >= lo) & (dst16 < lo + HALF)
      ldst16 = jnp.where(inh, dst16 - lo, jnp.bitwise_and(dst16, 4095))
      ldst_v[0, pl.ds(t * 16, 16)] = ldst16
      ex_h = []
      for h in range(H):
        av = plsc.load_gather(asrc_v, [src16 * 4 + h])
        bv = plsc.load_gather(adst_v, [ldst16 * 4 + h])
        al = av + bv
        al = jnp.where(al >= 0, al, 0.2 * al)
        ex = _vexp(al)
        ex = jnp.where(inh, ex, 0.0)
        den_idx[h, pl.ds(t * 16, 16)] = ldst16 * 4 + h
        den_val[h, pl.ds(t * 16, 16)] = ex
        ex_h.append(ex)
      exs.append(ex_h)

    for h in range(H):
      pltpu.sync_copy(den_val.at[h], den_sh.at[den_idx.at[h]], add=True)

    gat.wait()

    for t in range(G // 16):
      for l in range(16):
        e = t * 16 + l
        wv = [jnp.full((16,), exs[t][h][l]) for h in range(H)]
        for k in range(HC // 16):
          rows[e, pl.ds(k * 16, 16)] = \
              rows[e, pl.ds(k * 16, 16)] * wv[k // 2]
    pltpu.sync_copy(rows, agg_sh.at[ldst_v.at[0]], add=True)
    return carry

  lax.fori_loop(0, GPT, group, 0)
  plsc.subcore_barrier()
  pltpu.sync_copy(agg_sh.at[pl.ds(s * APC, APC)],
                  parts_hbm.at[c, pl.ds(s * APC, APC)])
  pltpu.sync_copy(den_sh.at[pl.ds(s * DPC, DPC)],
                  denp_hbm.at[c, pl.ds(s * DPC, DPC)])


def _run_edges(srcf, dst4d, asrc, adst, xh, zeros, zerosd):
  mesh = plsc.VectorSubcoreMesh(core_axis_name="c", subcore_axis_name="s")
  fn = pl.kernel(
      _edge_kernel,
      out_type=[
          jax.ShapeDtypeStruct((NCORES, HALF, HC), _f32),
          jax.ShapeDtypeStruct((NCORES, DLEN), _f32),
      ],
      mesh=mesh,
      compiler_params=pltpu.CompilerParams(needs_layout_passes=False),
      scratch_types=[
          pltpu.VMEM((4 * N,), _f32),
          pltpu.VMEM((4 * HALF,), _f32),
          pltpu.VMEM((SCH * G,), jnp.int32),
          pltpu.VMEM((SCH, G), jnp.int32),
          pltpu.VMEM((1, G), jnp.int32),
          pltpu.VMEM((H, G), jnp.int32),
          pltpu.VMEM((H, G), _f32),
          pltpu.VMEM((G, HC), _f32),
          pltpu.VMEM_SHARED((HALF, HC), _f32),
          pltpu.VMEM_SHARED((DLEN,), _f32),
          pltpu.SemaphoreType.DMA,
      ],
  )
  return fn(srcf, dst4d, asrc, adst, xh, zeros, zerosd)


def _head_kernel(parts_ref, denp_ref, batch_ref, b1_ref, wd1_ref, bd1_ref,
                 wd2_ref, bd2_ref, out_ref, gmax):
  i = pl.program_id(0)

  @pl.when(i == 0)
  def _():
    gmax[...] = jnp.full((NGRAPH, C), -1e30, _f32)

  agg = parts_ref[0]
  dsum = denp_ref[0]
  den_rep = jnp.concatenate(
      [jnp.broadcast_to(dsum[:, h:h + 1], (1024, C)) for h in range(H)],
      axis=1)

  h1 = agg / (den_rep + 1e-16) + b1_ref[...]
  h1 = jnp.where(h1 >= 0, h1, 0.01 * h1)
  h2 = jnp.dot(h1.astype(jnp.bfloat16), wd1_ref[...].astype(jnp.bfloat16),
               preferred_element_type=_f32) + bd1_ref[...]
  h2 = jnp.where(h2 >= 0, h2, 0.01 * h2)

  b = batch_ref[...]
  for g in range(NGRAPH):
    sel = jnp.where(b == g, h2, -1e30)
    m = jnp.max(sel, axis=0, keepdims=True)
    gmax[g:g + 1, :] = jnp.maximum(gmax[g:g + 1, :], m)

  @pl.when(i == pl.num_programs(0) - 1)
  def _():
    gf = gmax[...]
    gf = jnp.where(gf > -1e29, gf, 0.0)
    out_ref[...] = jnp.dot(gf.astype(jnp.bfloat16),
                           wd2_ref[...].astype(jnp.bfloat16),
                           preferred_element_type=_f32) + bd2_ref[...]


def _run_head(parts, denp, batchp, b1, Wd1, bd1, Wd2, bd2):
  bn = 1024
  nbh = HALF // bn
  return pl.pallas_call(
      _head_kernel,
      grid=(NPAD // bn,),
      in_specs=[
          pl.BlockSpec((1, bn, HC), lambda i: (i // nbh, i % nbh, 0)),
          pl.BlockSpec((1, bn, H), lambda i: (i // nbh, i % nbh, 0)),
          pl.BlockSpec((bn, 1), lambda i: (i, 0)),
          pl.BlockSpec((1, HC), lambda i: (0, 0)),
          pl.BlockSpec((HC, C), lambda i: (0, 0)),
          pl.BlockSpec((1, C), lambda i: (0, 0)),
          pl.BlockSpec((C, 1), lambda i: (0, 0)),
          pl.BlockSpec((1, 1), lambda i: (0, 0)),
      ],
      out_specs=pl.BlockSpec((NGRAPH, 1), lambda i: (0, 0)),
      out_shape=jax.ShapeDtypeStruct((NGRAPH, 1), _f32),
      scratch_shapes=[pltpu.VMEM((NGRAPH, C), _f32)],
  )(parts, denp, batchp, b1, Wd1, bd1, Wd2, bd2)


_RIDX = np.arange(HC)
_HIDX = _RIDX // C


def kernel(x, edge_index, edge_attr, batch, W1, att_src, att_dst, b1, Wd1,
           bd1, Wd2, bd2):
  del edge_attr
  src = edge_index[0]
  dst4d = edge_index[1].reshape(NSUB, NCHUNK, SCH, G)

  P = jnp.zeros((HC, 2 * H), _f32)
  P = P.at[_RIDX, _HIDX].set(att_src.reshape(HC))
  P = P.at[_RIDX, H + _HIDX].set(att_dst.reshape(HC))

  xh, asd = _run_proj(x, W1, P)
  asrc = asd[:, :H].reshape(H * N)
  adst = jnp.concatenate(
      [asd[:, H:], jnp.zeros((NPAD - N, H), _f32)]).reshape(NCORES, H * HALF)
  zeros = jnp.zeros((APC, HC), _f32)
  zerosd = jnp.zeros((DPC,), _f32)
  parts, denp = _run_edges(src, dst4d, asrc, adst, xh, zeros, zerosd)
  denp = denp.reshape(NCORES, HALF, H)
  batchp = jnp.concatenate(
      [batch, jnp.full((NPAD - N,), NGRAPH, jnp.int32)]).reshape(NPAD, 1)
  out = _run_head(parts, denp, batchp, b1.reshape(1, HC), Wd1,
                  bd1.reshape(1, C), Wd2, bd2.reshape(1, 1))
  return out


# speedup vs baseline: 1.4733x; 1.2510x over previous
import jax
import jax.numpy as jnp
import numpy as np
from jax import lax
from jax.experimental import pallas as pl
from jax.experimental.pallas import tpu as pltpu
from jax.experimental.pallas import tpu_sc as plsc

N = 10000
E = 320000
FIN = 128
H = 4
C = 32
HC = H * C
NGRAPH = 64

NCORES = 2
NSUB = 16
EPT = E // NSUB
G = 80
CHE = 2000
NCHUNK = EPT // CHE
CP16 = CHE // 16
CPAD = CHE + G
NPAD = 10240
HALF = NPAD // NCORES
APC = HALF // NSUB
DLEN = HALF * H
DPC = DLEN // NSUB

_f32 = jnp.float32


def _proj_kernel(x_ref, w1_ref, p_ref, xh_ref, asd_ref):
  xh = jnp.dot(x_ref[...].astype(jnp.bfloat16), w1_ref[...].astype(jnp.bfloat16),
               preferred_element_type=_f32)
  xh_ref[...] = xh
  asd_ref[...] = jnp.dot(xh, p_ref[...], preferred_element_type=_f32,
                         precision=lax.Precision.HIGHEST)


def _run_proj(x, W1, P):
  bn = 1000
  return pl.pallas_call(
      _proj_kernel,
      grid=(N // bn,),
      in_specs=[
          pl.BlockSpec((bn, FIN), lambda i: (i, 0)),
          pl.BlockSpec((FIN, HC), lambda i: (0, 0)),
          pl.BlockSpec((HC, 2 * H), lambda i: (0, 0)),
      ],
      out_specs=[
          pl.BlockSpec((bn, HC), lambda i: (i, 0)),
          pl.BlockSpec((bn, 2 * H), lambda i: (i, 0)),
      ],
      out_shape=[
          jax.ShapeDtypeStruct((N, HC), _f32),
          jax.ShapeDtypeStruct((N, 2 * H), _f32),
      ],
  )(x, W1, P)


_LOG2E = 1.4426950408889634
_LN2 = 0.6931471805599453


def _vexp(x):
  xl = x * _LOG2E
  xl = jnp.minimum(jnp.maximum(xl, -125.0), 125.0)
  half = jnp.where(xl >= 0, 0.5, -0.5)
  k = (xl + half).astype(jnp.int32)
  t = (xl - k.astype(_f32)) * _LN2
  p = 1.0 + t * (1.0 + t * (0.5 + t * (
      0.16666666666666666 + t * (0.041666666666666664 + t * (
          0.008333333333333333 + t * 0.001388888888888889)))))
  return p * plsc.bitcast(lax.shift_left(k + 127, 23), _f32)


def _edge_kernel(srcf_hbm, dstf_hbm, asrc_hbm, adst_hbm, xh_hbm, zeros_hbm,
                 zerosd_hbm, parts_hbm, denp_hbm,
                 asrc_v, adst_v, srcs_v, dsts_v, csrc_v, cdst_v, ldst_v,
                 den_idx, den_val, rows, agg_sh, den_sh, sem):
  c = lax.axis_index("c")
  s = lax.axis_index("s")
  lo = c * HALF

  pltpu.sync_copy(asrc_hbm, asrc_v)
  pltpu.sync_copy(adst_hbm.at[c], adst_v)
  pltpu.sync_copy(zeros_hbm, agg_sh.at[pl.ds(s * APC, APC)])
  pltpu.sync_copy(zerosd_hbm, den_sh.at[pl.ds(s * DPC, DPC)])
  zero16i = jnp.zeros((16,), jnp.int32)

  def zinit(t, carry):
    csrc_v[pl.ds(t * 16, 16)] = zero16i
    cdst_v[pl.ds(t * 16, 16)] = zero16i
    return carry

  lax.fori_loop(0, CPAD // 16, zinit, 0)
  plsc.subcore_barrier()

  iota16 = lax.iota(jnp.int32, 16)

  def chunk(ch, carry0):
    pltpu.sync_copy(srcf_hbm.at[pl.ds(s * EPT + ch * CHE, CHE)], srcs_v)
    pltpu.sync_copy(dstf_hbm.at[pl.ds(s * EPT + ch * CHE, CHE)], dsts_v)

    def compact(t, cnt):
      s16 = srcs_v[pl.ds(t * 16, 16)]
      d16 = dsts_v[pl.ds(t * 16, 16)]
      m = (d16 >= lo) & (d16 < lo + HALF)
      plsc.store_compressed(csrc_v.at[pl.ds(cnt, 16)], s16, mask=m)
      plsc.store_compressed(cdst_v.at[pl.ds(cnt, 16)], d16 - lo, mask=m)
      return cnt + plsc.all_reduce_population_count(m)[0]

    cnt = lax.fori_loop(0, CP16, compact, 0, unroll=2)
    ngroups = (cnt + (G - 1)) // G

    def group(g, carry):
      base = g * G
      gat = pltpu.async_copy(xh_hbm.at[csrc_v.at[pl.ds(base, G)]], rows, sem)

      exs = []
      for t in range(G // 16):
        src16 = csrc_v[pl.ds(base + t * 16, 16)]
        ldst16 = cdst_v[pl.ds(base + t * 16, 16)]
        valid = base + t * 16 + iota16 < cnt
        ldst_v[0, pl.ds(t * 16, 16)] = ldst16
        ex_h = []
        for h in range(H):
          av = plsc.load_gather(asrc_v, [src16 * 4 + h])
          bv = plsc.load_gather(adst_v, [ldst16 * 4 + h])
          al = av + bv
          al = jnp.where(al >= 0, al, 0.2 * al)
          ex = _vexp(al)
          ex = jnp.where(valid, ex, 0.0)
          den_idx[h, pl.ds(t * 16, 16)] = ldst16 * 4 + h
          den_val[h, pl.ds(t * 16, 16)] = ex
          ex_h.append(ex)
        exs.append(ex_h)

      for h in range(H):
        pltpu.sync_copy(den_val.at[h], den_sh.at[den_idx.at[h]], add=True)

      gat.wait()

      for t in range(G // 16):
        for l in range(16):
          e = t * 16 + l
          wv = [jnp.full((16,), exs[t][h][l]) for h in range(H)]
          for k in range(HC // 16):
            rows[e, pl.ds(k * 16, 16)] = \
                rows[e, pl.ds(k * 16, 16)] * wv[k // 2]
      pltpu.sync_copy(rows, agg_sh.at[ldst_v.at[0]], add=True)
      return carry

    lax.fori_loop(0, ngroups, group, 0)
    return carry0

  lax.fori_loop(0, NCHUNK, chunk, 0)
  plsc.subcore_barrier()
  pltpu.sync_copy(agg_sh.at[pl.ds(s * APC, APC)],
                  parts_hbm.at[c, pl.ds(s * APC, APC)])
  pltpu.sync_copy(den_sh.at[pl.ds(s * DPC, DPC)],
                  denp_hbm.at[c, pl.ds(s * DPC, DPC)])


def _run_edges(srcf, dstf, asrc, adst, xh, zeros, zerosd):
  mesh = plsc.VectorSubcoreMesh(core_axis_name="c", subcore_axis_name="s")
  fn = pl.kernel(
      _edge_kernel,
      out_type=[
          jax.ShapeDtypeStruct((NCORES, HALF, HC), _f32),
          jax.ShapeDtypeStruct((NCORES, DLEN), _f32),
      ],
      mesh=mesh,
      compiler_params=pltpu.CompilerParams(needs_layout_passes=False),
      scratch_types=[
          pltpu.VMEM((4 * N,), _f32),
          pltpu.VMEM((4 * HALF,), _f32),
          pltpu.VMEM((CHE,), jnp.int32),
          pltpu.VMEM((CHE,), jnp.int32),
          pltpu.VMEM((CPAD,), jnp.int32),
          pltpu.VMEM((CPAD,), jnp.int32),
          pltpu.VMEM((1, G), jnp.int32),
          pltpu.VMEM((H, G), jnp.int32),
          pltpu.VMEM((H, G), _f32),
          pltpu.VMEM((G, HC), _f32),
          pltpu.VMEM_SHARED((HALF, HC), _f32),
          pltpu.VMEM_SHARED((DLEN,), _f32),
          pltpu.SemaphoreType.DMA,
      ],
  )
  return fn(srcf, dstf, asrc, adst, xh, zeros, zerosd)


def _head_kernel(parts_ref, denp_ref, batch_ref, b1_ref, wd1_ref, bd1_ref,
                 wd2_ref, bd2_ref, out_ref, gmax):
  i = pl.program_id(0)

  @pl.when(i == 0)
  def _():
    gmax[...] = jnp.full((NGRAPH, C), -1e30, _f32)

  agg = parts_ref[0]
  dsum = denp_ref[0]
  den_rep = jnp.concatenate(
      [jnp.broadcast_to(dsum[:, h:h + 1], (1024, C)) for h in range(H)],
      axis=1)

  h1 = agg / (den_rep + 1e-16) + b1_ref[...]
  h1 = jnp.where(h1 >= 0, h1, 0.01 * h1)
  h2 = jnp.dot(h1.astype(jnp.bfloat16), wd1_ref[...].astype(jnp.bfloat16),
               preferred_element_type=_f32) + bd1_ref[...]
  h2 = jnp.where(h2 >= 0, h2, 0.01 * h2)

  b = batch_ref[...]
  for g in range(NGRAPH):
    sel = jnp.where(b == g, h2, -1e30)
    m = jnp.max(sel, axis=0, keepdims=True)
    gmax[g:g + 1, :] = jnp.maximum(gmax[g:g + 1, :], m)

  @pl.when(i == pl.num_programs(0) - 1)
  def _():
    gf = gmax[...]
    gf = jnp.where(gf > -1e29, gf, 0.0)
    out_ref[...] = jnp.dot(gf.astype(jnp.bfloat16),
                           wd2_ref[...].astype(jnp.bfloat16),
                           preferred_element_type=_f32) + bd2_ref[...]


def _run_head(parts, denp, batchp, b1, Wd1, bd1, Wd2, bd2):
  bn = 1024
  nbh = HALF // bn
  return pl.pallas_call(
      _head_kernel,
      grid=(NPAD // bn,),
      in_specs=[
          pl.BlockSpec((1, bn, HC), lambda i: (i // nbh, i % nbh, 0)),
          pl.BlockSpec((1, bn, H), lambda i: (i // nbh, i % nbh, 0)),
          pl.BlockSpec((bn, 1), lambda i: (i, 0)),
          pl.BlockSpec((1, HC), lambda i: (0, 0)),
          pl.BlockSpec((HC, C), lambda i: (0, 0)),
          pl.BlockSpec((1, C), lambda i: (0, 0)),
          pl.BlockSpec((C, 1), lambda i: (0, 0)),
          pl.BlockSpec((1, 1), lambda i: (0, 0)),
      ],
      out_specs=pl.BlockSpec((NGRAPH, 1), lambda i: (0, 0)),
      out_shape=jax.ShapeDtypeStruct((NGRAPH, 1), _f32),
      scratch_shapes=[pltpu.VMEM((NGRAPH, C), _f32)],
  )(parts, denp, batchp, b1, Wd1, bd1, Wd2, bd2)


_RIDX = np.arange(HC)
_HIDX = _RIDX // C


def kernel(x, edge_index, edge_attr, batch, W1, att_src, att_dst, b1, Wd1,
           bd1, Wd2, bd2):
  del edge_attr
  src = edge_index[0]
  dst = edge_index[1]

  P = jnp.zeros((HC, 2 * H), _f32)
  P = P.at[_RIDX, _HIDX].set(att_src.reshape(HC))
  P = P.at[_RIDX, H + _HIDX].set(att_dst.reshape(HC))

  xh, asd = _run_proj(x, W1, P)
  asrc = asd[:, :H].reshape(H * N)
  adst = jnp.concatenate(
      [asd[:, H:], jnp.zeros((NPAD - N, H), _f32)]).reshape(NCORES, H * HALF)
  zeros = jnp.zeros((APC, HC), _f32)
  zerosd = jnp.zeros((DPC,), _f32)
  parts, denp = _run_edges(src, dst, asrc, adst, xh, zeros, zerosd)
  denp = denp.reshape(NCORES, HALF, H)
  batchp = jnp.concatenate(
      [batch, jnp.full((NPAD - N,), NGRAPH, jnp.int32)]).reshape(NPAD, 1)
  out = _run_head(parts, denp, batchp, b1.reshape(1, HC), Wd1,
                  bd1.reshape(1, C), Wd2, bd2.reshape(1, 1))
  return out


# speedup vs baseline: 1.6485x; 1.1189x over previous
import jax
import jax.numpy as jnp
import numpy as np
from jax import lax
from jax.experimental import pallas as pl
from jax.experimental.pallas import tpu as pltpu
from jax.experimental.pallas import tpu_sc as plsc

N = 10000
E = 320000
FIN = 128
H = 4
C = 32
HC = H * C
NGRAPH = 64

NCORES = 2
NSUB = 16
EPT = E // NSUB
G = 80
CHE = 2000
NCHUNK = EPT // CHE
CP16 = CHE // 16
CPAD = CHE + G
NPAD = 10240
HALF = NPAD // NCORES
APC = HALF // NSUB
DLEN = HALF * H
DPC = DLEN // NSUB

_f32 = jnp.float32


def _proj_kernel(x_ref, w1_ref, p_ref, xh_ref, asd_ref):
  xh = jnp.dot(x_ref[...].astype(jnp.bfloat16), w1_ref[...].astype(jnp.bfloat16),
               preferred_element_type=_f32)
  xh_ref[...] = xh
  asd_ref[...] = jnp.dot(xh, p_ref[...], preferred_element_type=_f32,
                         precision=lax.Precision.HIGHEST)


def _run_proj(x, W1, P):
  bn = 1000
  return pl.pallas_call(
      _proj_kernel,
      grid=(N // bn,),
      in_specs=[
          pl.BlockSpec((bn, FIN), lambda i: (i, 0)),
          pl.BlockSpec((FIN, HC), lambda i: (0, 0)),
          pl.BlockSpec((HC, 2 * H), lambda i: (0, 0)),
      ],
      out_specs=[
          pl.BlockSpec((bn, HC), lambda i: (i, 0)),
          pl.BlockSpec((bn, 2 * H), lambda i: (i, 0)),
      ],
      out_shape=[
          jax.ShapeDtypeStruct((N, HC), _f32),
          jax.ShapeDtypeStruct((N, 2 * H), _f32),
      ],
  )(x, W1, P)


_LOG2E = 1.4426950408889634
_LN2 = 0.6931471805599453


def _vexp(x):
  xl = x * _LOG2E
  xl = jnp.minimum(jnp.maximum(xl, -125.0), 125.0)
  half = jnp.where(xl >= 0, 0.5, -0.5)
  k = (xl + half).astype(jnp.int32)
  t = (xl - k.astype(_f32)) * _LN2
  p = 1.0 + t * (1.0 + t * (0.5 + t * (
      0.16666666666666666 + t * (0.041666666666666664 + t * (
          0.008333333333333333 + t * 0.001388888888888889)))))
  return p * plsc.bitcast(lax.shift_left(k + 127, 23), _f32)


def _edge_kernel(srcf_hbm, dstf_hbm, asrc_hbm, adst_hbm, xh_hbm, zeros_hbm,
                 zerosd_hbm, parts_hbm, denp_hbm,
                 asrc_v, adst_v, srcs_v, dsts_v, csrc_v, cdst_v, ldst_v,
                 den_idx, den_val, rows, agg_sh, den_sh, sem):
  c = lax.axis_index("c")
  s = lax.axis_index("s")
  lo = c * HALF

  pltpu.sync_copy(asrc_hbm, asrc_v)
  pltpu.sync_copy(adst_hbm.at[c], adst_v)
  pltpu.sync_copy(zeros_hbm, agg_sh.at[pl.ds(s * APC, APC)])
  pltpu.sync_copy(zerosd_hbm, den_sh.at[pl.ds(s * DPC, DPC)])
  zero16i = jnp.zeros((16,), jnp.int32)

  def zinit(t, carry):
    csrc_v[pl.ds(t * 16, 16)] = zero16i
    cdst_v[pl.ds(t * 16, 16)] = zero16i
    return carry

  lax.fori_loop(0, CPAD // 16, zinit, 0)
  plsc.subcore_barrier()

  iota16 = lax.iota(jnp.int32, 16)

  def chunk(ch, carry0):
    pltpu.sync_copy(srcf_hbm.at[pl.ds(s * EPT + ch * CHE, CHE)], srcs_v)
    pltpu.sync_copy(dstf_hbm.at[pl.ds(s * EPT + ch * CHE, CHE)], dsts_v)

    def compact(t, cnt):
      s16 = srcs_v[pl.ds(t * 16, 16)]
      d16 = dsts_v[pl.ds(t * 16, 16)]
      m = (d16 >= lo) & (d16 < lo + HALF)
      plsc.store_compressed(csrc_v.at[pl.ds(cnt, 16)], s16, mask=m)
      plsc.store_compressed(cdst_v.at[pl.ds(cnt, 16)], d16 - lo, mask=m)
      return cnt + plsc.all_reduce_population_count(m)[0]

    cnt = lax.fori_loop(0, CP16, compact, 0, unroll=2)
    ngroups = (cnt + (G - 1)) // G

    def group(g, carry):
      base = g * G
      gat = pltpu.async_copy(xh_hbm.at[csrc_v.at[pl.ds(base, G)]], rows, sem)

      exs = []
      for t in range(G // 16):
        src16 = csrc_v[pl.ds(base + t * 16, 16)]
        ldst16 = cdst_v[pl.ds(base + t * 16, 16)]
        valid = base + t * 16 + iota16 < cnt
        ldst_v[0, pl.ds(t * 16, 16)] = ldst16
        ex_h = []
        for h in range(H):
          av = plsc.load_gather(asrc_v, [src16 * 4 + h])
          bv = plsc.load_gather(adst_v, [ldst16 * 4 + h])
          al = av + bv
          al = jnp.where(al >= 0, al, 0.2 * al)
          ex = _vexp(al)
          ex = jnp.where(valid, ex, 0.0)
          den_idx[h, pl.ds(t * 16, 16)] = ldst16 * 4 + h
          den_val[h, pl.ds(t * 16, 16)] = ex
          ex_h.append(ex)
        exs.append(ex_h)

      for h in range(H):
        pltpu.sync_copy(den_val.at[h], den_sh.at[den_idx.at[h]], add=True)

      gat.wait()

      for t in range(G // 16):
        for l in range(16):
          e = t * 16 + l
          wv = [jnp.full((16,), exs[t][h][l]) for h in range(H)]
          for k in range(HC // 16):
            rows[e, pl.ds(k * 16, 16)] = \
                rows[e, pl.ds(k * 16, 16)] * wv[k // 2]
      pltpu.sync_copy(rows, agg_sh.at[ldst_v.at[0]], add=True)
      return carry

    lax.fori_loop(0, ngroups, group, 0)
    return carry0

  lax.fori_loop(0, NCHUNK, chunk, 0)
  plsc.subcore_barrier()
  pltpu.sync_copy(agg_sh.at[pl.ds(s * APC, APC)],
                  parts_hbm.at[c, pl.ds(s * APC, APC)])
  pltpu.sync_copy(den_sh.at[pl.ds(s * DPC, DPC)],
                  denp_hbm.at[c, pl.ds(s * DPC, DPC)])


def _run_edges(srcf, dstf, asrc, adst, xh, zeros, zerosd):
  mesh = plsc.VectorSubcoreMesh(core_axis_name="c", subcore_axis_name="s")
  fn = pl.kernel(
      _edge_kernel,
      out_type=[
          jax.ShapeDtypeStruct((NCORES, HALF, HC), _f32),
          jax.ShapeDtypeStruct((NCORES, DLEN), _f32),
      ],
      mesh=mesh,
      compiler_params=pltpu.CompilerParams(needs_layout_passes=False),
      scratch_types=[
          pltpu.VMEM((4 * N,), _f32),
          pltpu.VMEM((4 * HALF,), _f32),
          pltpu.VMEM((CHE,), jnp.int32),
          pltpu.VMEM((CHE,), jnp.int32),
          pltpu.VMEM((CPAD,), jnp.int32),
          pltpu.VMEM((CPAD,), jnp.int32),
          pltpu.VMEM((1, G), jnp.int32),
          pltpu.VMEM((H, G), jnp.int32),
          pltpu.VMEM((H, G), _f32),
          pltpu.VMEM((G, HC), _f32),
          pltpu.VMEM_SHARED((HALF, HC), _f32),
          pltpu.VMEM_SHARED((DLEN,), _f32),
          pltpu.SemaphoreType.DMA,
      ],
  )
  return fn(srcf, dstf, asrc, adst, xh, zeros, zerosd)


def _head_kernel(parts_ref, denp_ref, batch_ref, b1_ref, wd1_ref, bd1_ref,
                 wd2_ref, bd2_ref, out_ref, gmax):
  i = pl.program_id(0)

  @pl.when(i == 0)
  def _():
    gmax[...] = jnp.full((NGRAPH, C), -1e30, _f32)

  agg = parts_ref[0]
  dsum = denp_ref[0]
  den_rep = jnp.concatenate(
      [jnp.broadcast_to(dsum[:, h:h + 1], (1024, C)) for h in range(H)],
      axis=1)

  h1 = agg / (den_rep + 1e-16) + b1_ref[...]
  h1 = jnp.where(h1 >= 0, h1, 0.01 * h1)
  h2 = jnp.dot(h1.astype(jnp.bfloat16), wd1_ref[...].astype(jnp.bfloat16),
               preferred_element_type=_f32) + bd1_ref[...]
  h2 = jnp.where(h2 >= 0, h2, 0.01 * h2)

  b = batch_ref[...]
  glo = jnp.min(b)
  ghi = jnp.minimum(jnp.max(b), NGRAPH - 1)

  def upd(g, carry):
    sel = jnp.where(b == g, h2, -1e30)
    m = jnp.max(sel, axis=0, keepdims=True)
    gmax[pl.ds(g, 1), :] = jnp.maximum(gmax[pl.ds(g, 1), :], m)
    return carry

  lax.fori_loop(glo, ghi + 1, upd, 0)

  @pl.when(i == pl.num_programs(0) - 1)
  def _():
    gf = gmax[...]
    gf = jnp.where(gf > -1e29, gf, 0.0)
    out_ref[...] = jnp.dot(gf.astype(jnp.bfloat16),
                           wd2_ref[...].astype(jnp.bfloat16),
                           preferred_element_type=_f32) + bd2_ref[...]


def _run_head(parts, denp, batchp, b1, Wd1, bd1, Wd2, bd2):
  bn = 1024
  nbh = HALF // bn
  return pl.pallas_call(
      _head_kernel,
      grid=(NPAD // bn,),
      in_specs=[
          pl.BlockSpec((1, bn, HC), lambda i: (i // nbh, i % nbh, 0)),
          pl.BlockSpec((1, bn, H), lambda i: (i // nbh, i % nbh, 0)),
          pl.BlockSpec((bn, 1), lambda i: (i, 0)),
          pl.BlockSpec((1, HC), lambda i: (0, 0)),
          pl.BlockSpec((HC, C), lambda i: (0, 0)),
          pl.BlockSpec((1, C), lambda i: (0, 0)),
          pl.BlockSpec((C, 1), lambda i: (0, 0)),
          pl.BlockSpec((1, 1), lambda i: (0, 0)),
      ],
      out_specs=pl.BlockSpec((NGRAPH, 1), lambda i: (0, 0)),
      out_shape=jax.ShapeDtypeStruct((NGRAPH, 1), _f32),
      scratch_shapes=[pltpu.VMEM((NGRAPH, C), _f32)],
  )(parts, denp, batchp, b1, Wd1, bd1, Wd2, bd2)


_RIDX = np.arange(HC)
_HIDX = _RIDX // C


def kernel(x, edge_index, edge_attr, batch, W1, att_src, att_dst, b1, Wd1,
           bd1, Wd2, bd2):
  del edge_attr
  src = edge_index[0]
  dst = edge_index[1]

  P = jnp.zeros((HC, 2 * H), _f32)
  P = P.at[_RIDX, _HIDX].set(att_src.reshape(HC))
  P = P.at[_RIDX, H + _HIDX].set(att_dst.reshape(HC))

  xh, asd = _run_proj(x, W1, P)
  asrc = asd[:, :H].reshape(H * N)
  adst = jnp.concatenate(
      [asd[:, H:], jnp.zeros((NPAD - N, H), _f32)]).reshape(NCORES, H * HALF)
  zeros = jnp.zeros((APC, HC), _f32)
  zerosd = jnp.zeros((DPC,), _f32)
  parts, denp = _run_edges(src, dst, asrc, adst, xh, zeros, zerosd)
  denp = denp.reshape(NCORES, HALF, H)
  batchp = jnp.concatenate(
      [batch, jnp.full((NPAD - N,), NGRAPH, jnp.int32)]).reshape(NPAD, 1)
  out = _run_head(parts, denp, batchp, b1.reshape(1, HC), Wd1,
                  bd1.reshape(1, C), Wd2, bd2.reshape(1, 1))
  return out
